# baseline (speedup 1.0000x reference)
import jax
import jax.numpy as jnp
from jax.experimental import pallas as pl
from jax.experimental.pallas import tpu as pltpu

N = 208
NFEAT = 512
NHID = 256
NCLASS = 128
NCHUNK = 16
ROWS = N // NCHUNK
CHUNK = ROWS * NCLASS


def _fused(x_ref, adj_ref, w1_ref, b1_ref, w2_ref, b2_ref,
           fc1w_ref, fc1b_ref, fc2w_ref, fc2b_ref, out_ref,
           h2_ref, acc_ref):
    i = pl.program_id(0)

    @pl.when(i == 0)
    def _gcn():
        adj = adj_ref[...]
        t1 = jnp.dot(x_ref[...], w1_ref[...],
                     preferred_element_type=jnp.float32)
        h1 = jnp.maximum(jnp.dot(adj, t1, preferred_element_type=jnp.float32)
                         + b1_ref[...], 0.0)
        t2 = jnp.dot(h1, w2_ref[...], preferred_element_type=jnp.float32)
        h2 = jnp.maximum(jnp.dot(adj, t2, preferred_element_type=jnp.float32)
                         + b2_ref[...], 0.0)
        h2_ref[...] = h2.reshape(NCHUNK, ROWS, NCLASS)
        acc_ref[...] = jnp.zeros((1, 60), jnp.float32)

    flat = h2_ref[i].reshape(1, CHUNK)
    acc_ref[...] += jax.lax.dot_general(flat, fc1w_ref[...],
                                        (((1,), (1,)), ((), ())),
                                        preferred_element_type=jnp.float32)

    @pl.when(i == NCHUNK - 1)
    def _head():
        h3 = jnp.maximum(acc_ref[...] + fc1b_ref[...], 0.0)
        z = jnp.sum(h3 * fc2w_ref[...], axis=1, keepdims=True)
        out_ref[...] = jax.nn.sigmoid(z + fc2b_ref[0, 0])


def kernel(x, adj, W1, b1, W2, b2, fc1_w, fc1_b, fc2_w, fc2_b):
    out = pl.pallas_call(
        _fused,
        grid=(NCHUNK,),
        in_specs=[
            pl.BlockSpec((N, NFEAT), lambda i: (0, 0)),
            pl.BlockSpec((N, N), lambda i: (0, 0)),
            pl.BlockSpec((NFEAT, NHID), lambda i: (0, 0)),
            pl.BlockSpec((1, NHID), lambda i: (0, 0)),
            pl.BlockSpec((NHID, NCLASS), lambda i: (0, 0)),
            pl.BlockSpec((1, NCLASS), lambda i: (0, 0)),
            pl.BlockSpec((60, CHUNK), lambda i: (0, i)),
            pl.BlockSpec((1, 60), lambda i: (0, 0)),
            pl.BlockSpec((1, 60), lambda i: (0, 0)),
            pl.BlockSpec(memory_space=pltpu.SMEM),
        ],
        out_specs=pl.BlockSpec((1, 1), lambda i: (0, 0)),
        out_shape=jax.ShapeDtypeStruct((1, 1), jnp.float32),
        scratch_shapes=[
            pltpu.VMEM((NCHUNK, ROWS, NCLASS), jnp.float32),
            pltpu.VMEM((1, 60), jnp.float32),
        ],
        compiler_params=pltpu.CompilerParams(
            dimension_semantics=("arbitrary",),
        ),
    )(x, adj, W1, b1.reshape(1, NHID), W2, b2.reshape(1, NCLASS),
      fc1_w, fc1_b.reshape(1, 60), fc2_w, fc2_b.reshape(1, 1))
    return out.reshape(1)


# speedup vs baseline: 1.9051x; 1.9051x over previous
import jax
import jax.numpy as jnp
from jax.experimental import pallas as pl
from jax.experimental.pallas import tpu as pltpu

N = 208
NFEAT = 512
NHID = 256
NCLASS = 128


def _fused(x_ref, adj_ref, w1_ref, b1_ref, w2_ref, b2_ref,
           fc1w_hbm, fc1b_ref, fc2w_ref, fc2b_ref, out_ref,
           fc1w_vmem, dma_sem):
    cp = pltpu.make_async_copy(fc1w_hbm, fc1w_vmem, dma_sem)
    cp.start()
    adj = adj_ref[...]
    t1 = jnp.dot(x_ref[...], w1_ref[...], preferred_element_type=jnp.float32)
    h1 = jnp.maximum(jnp.dot(adj, t1, preferred_element_type=jnp.float32)
                     + b1_ref[...], 0.0)
    t2 = jnp.dot(h1, w2_ref[...], preferred_element_type=jnp.float32)
    h2 = jnp.maximum(jnp.dot(adj, t2, preferred_element_type=jnp.float32)
                     + b2_ref[...], 0.0)
    flat = h2.reshape(1, N * NCLASS)
    cp.wait()
    h3 = jax.lax.dot_general(flat, fc1w_vmem[...],
                             (((1,), (1,)), ((), ())),
                             preferred_element_type=jnp.float32)
    h3 = jnp.maximum(h3 + fc1b_ref[...], 0.0)
    z = jnp.sum(h3 * fc2w_ref[...], axis=1, keepdims=True)
    out_ref[...] = jax.nn.sigmoid(z + fc2b_ref[0, 0])


def kernel(x, adj, W1, b1, W2, b2, fc1_w, fc1_b, fc2_w, fc2_b):
    out = pl.pallas_call(
        _fused,
        out_shape=jax.ShapeDtypeStruct((1, 1), jnp.float32),
        in_specs=[pl.BlockSpec(memory_space=pltpu.VMEM)] * 6
                 + [pl.BlockSpec(memory_space=pl.ANY)]
                 + [pl.BlockSpec(memory_space=pltpu.VMEM)] * 2
                 + [pl.BlockSpec(memory_space=pltpu.SMEM)],
        out_specs=pl.BlockSpec(memory_space=pltpu.VMEM),
        scratch_shapes=[
            pltpu.VMEM((60, N * NCLASS), jnp.float32),
            pltpu.SemaphoreType.DMA,
        ],
    )(x, adj, W1, b1.reshape(1, NHID), W2, b2.reshape(1, NCLASS),
      fc1_w, fc1_b.reshape(1, 60), fc2_w, fc2_b.reshape(1, 1))
    return out.reshape(1)
